# baseline (speedup 1.0000x reference)
import functools

import jax
import jax.numpy as jnp
from jax import lax
from jax.experimental import pallas as pl
from jax.experimental.pallas import tpu as pltpu
from jax.experimental.pallas import tpu_sc as plsc

EMBED_DIM = 64
PAD_DIM = 128
CHUNK = 128
NBUF = 4


@functools.cache
def _make_kernel(n_out: int):
    info = plsc.get_sparse_core_info()
    num_cores = info.num_cores
    num_workers = info.num_cores * info.num_subcores
    b_per_w = n_out // num_workers
    assert n_out % num_workers == 0 and b_per_w % CHUNK == 0
    n_chunks = b_per_w // CHUNK
    n_groups = n_chunks // NBUF
    assert n_chunks % NBUF == 0 and n_groups >= 3

    mesh = plsc.VectorSubcoreMesh(core_axis_name="c", subcore_axis_name="s")

    @functools.partial(
        pl.kernel,
        mesh=mesh,
        out_type=jax.ShapeDtypeStruct((n_out, PAD_DIM), jnp.float32),
        scratch_types=[
            pltpu.VMEM((b_per_w,), jnp.int32),
            pltpu.VMEM((b_per_w,), jnp.int32),
            pltpu.VMEM((NBUF, CHUNK, PAD_DIM), jnp.float32),
            pltpu.SemaphoreType.DMA,
            pltpu.SemaphoreType.DMA,
            pltpu.SemaphoreType.DMA,
        ],
    )
    def emb_kernel(idx_e_hbm, idx_o_hbm, tab_l_hbm, tab_r_hbm, out_hbm,
                   idx_e_v, idx_o_v, buf, gl_sem, gr_sem, o_sem):
        wid = lax.axis_index("s") * num_cores + lax.axis_index("c")
        base = wid * b_per_w
        pltpu.sync_copy(idx_e_hbm.at[pl.ds(base, b_per_w)], idx_e_v)
        pltpu.sync_copy(idx_o_hbm.at[pl.ds(base, b_per_w)], idx_o_v)

        def start_gl(i, b):
            off = pl.multiple_of(i * CHUNK, CHUNK)
            pltpu.async_copy(
                tab_l_hbm.at[idx_e_v.at[pl.ds(off, CHUNK)]], buf.at[b],
                gl_sem)

        def wait_gl(b):
            pltpu.make_async_copy(
                tab_l_hbm.at[idx_e_v.at[pl.ds(0, CHUNK)]], buf.at[b],
                gl_sem).wait()

        def start_gr(i, b):
            off = pl.multiple_of(i * CHUNK, CHUNK)
            pltpu.async_copy(
                tab_r_hbm.at[idx_o_v.at[pl.ds(off, CHUNK)]], buf.at[b],
                gr_sem, add=True)

        def wait_gr(b):
            pltpu.make_async_copy(
                tab_r_hbm.at[idx_o_v.at[pl.ds(0, CHUNK)]], buf.at[b],
                gr_sem).wait()

        def start_out(i, b):
            off = pl.multiple_of(base + i * CHUNK, CHUNK)
            pltpu.async_copy(buf.at[b], out_hbm.at[pl.ds(off, CHUNK)], o_sem)

        def wait_out(b):
            pltpu.make_async_copy(
                buf.at[b], out_hbm.at[pl.ds(0, CHUNK)], o_sem).wait()

        def visit(i, b, retire_prev=True, start_next=True):
            wait_gl(b)
            start_gr(i, b)
            if retire_prev:
                wait_out((b - 1) % NBUF)
            if start_next:
                start_gl(i - 1 + NBUF, (b - 1) % NBUF)
            wait_gr(b)
            start_out(i, b)

        for b in range(NBUF):
            start_gl(b, b)

        for b in range(NBUF):
            visit(b, b, retire_prev=b >= 1, start_next=b >= 1)

        @pl.loop(1, n_groups - 1)
        def _(t):
            for b in range(NBUF):
                visit(t * NBUF + b, b, start_next=True)

        for b in range(NBUF):
            i = (n_groups - 1) * NBUF + b
            visit(i, b, start_next=(i - 1 + NBUF) < n_chunks)

        wait_out((n_chunks - 1) % NBUF)

    return emb_kernel


@jax.jit
def kernel(input_id, table):
    batch, seq_len = input_id.shape
    n_idx = batch * seq_len
    n_out = n_idx // 2
    flat_idx = input_id.reshape(n_idx)
    idx_even = flat_idx[0::2]
    idx_odd = flat_idx[1::2]
    tab_left = jnp.pad(table, ((0, 0), (0, PAD_DIM - EMBED_DIM)))
    tab_right = jnp.pad(table, ((0, 0), (PAD_DIM - EMBED_DIM, 0)))
    out = _make_kernel(n_out)(idx_even, idx_odd, tab_left, tab_right)
    return out.reshape(batch, seq_len, EMBED_DIM)
